# baseline (speedup 1.0000x reference)
import functools

import jax
import jax.numpy as jnp
from jax import lax
from jax.experimental import pallas as pl
from jax.experimental.pallas import tpu as pltpu
from jax.experimental.pallas import tpu_sc as plsc

_B, _F, _V = 16384, 26, 100000
_DE, _DH = 16, 32
_N = _B * _F
_NW = 32
_NPW = _N // _NW
_CH = 1024
_NCH = _NPW // _CH
_L = 16


def _ser_body(x_hbm, easy_hbm, hard_hbm, easy_out, hard_out,
              idx_v, easy_b, hard_b, sem_g, sem_o):
    wid = lax.axis_index("s") * 2 + lax.axis_index("c")
    base = wid * _NPW

    pltpu.sync_copy(x_hbm.at[pl.ds(base, _NPW)], idx_v)

    fld0 = lax.rem(base + lax.iota(jnp.int32, _L), _F)

    def fix(i, fld):
        sl = pl.ds(i * _L, _L)
        idx_v[sl] = idx_v[sl] + fld * _V
        nxt = fld + _L
        return jnp.where(nxt >= _F, nxt - _F, nxt)

    lax.fori_loop(0, _NPW // _L, fix, fld0)

    def step(c, carry):
        sl = pl.ds(c * _CH, _CH)
        out_sl = pl.ds(base + c * _CH, _CH)
        ce = pltpu.async_copy(easy_hbm.at[idx_v.at[sl]], easy_b, sem_g)
        ch = pltpu.async_copy(hard_hbm.at[idx_v.at[sl]], hard_b, sem_g)
        ce.wait()
        ch.wait()
        pltpu.sync_copy(easy_b, easy_out.at[out_sl])
        pltpu.sync_copy(hard_b, hard_out.at[out_sl])
        return carry

    lax.fori_loop(0, _NCH, step, 0)


@jax.jit
def _ser(x_flat, easy_flat, hard_flat):
    mesh = plsc.VectorSubcoreMesh(core_axis_name="c", subcore_axis_name="s")
    return pl.kernel(
        _ser_body,
        out_type=(
            jax.ShapeDtypeStruct((_N, _DE), jnp.float32),
            jax.ShapeDtypeStruct((_N, _DH), jnp.float32),
        ),
        mesh=mesh,
        scratch_types=[
            pltpu.VMEM((_NPW,), jnp.int32),
            pltpu.VMEM((_CH, _DE), jnp.float32),
            pltpu.VMEM((_CH, _DH), jnp.float32),
            pltpu.SemaphoreType.DMA,
            pltpu.SemaphoreType.DMA,
        ],
        compiler_params=pltpu.CompilerParams(use_tc_tiling_on_sc=False),
    )(x_flat, easy_flat, hard_flat)


def kernel(X, easy_table, hard_table):
    x_flat = X.reshape(_N)
    easy_flat = easy_table.reshape(_F * _V, _DE)
    hard_flat = hard_table.reshape(_F * _V, _DH)
    easy_rows, hard_rows = _ser(x_flat, easy_flat, hard_flat)
    return (easy_rows.reshape(_B, _F * _DE), hard_rows.reshape(_B, _F * _DH))
